# baseline (speedup 1.0000x reference)
import functools

import jax
import jax.numpy as jnp
from jax import lax
from jax.experimental import pallas as pl
from jax.experimental.pallas import tpu as pltpu
from jax.experimental.pallas import tpu_sc as plsc

_K = 300
_Q = 900
_C = 92
_N = _Q * _C
_KPAD = 304
_CAP = 2048
_NBIN1 = 544
_NBIN2 = 256
_LANES = 16
_NPAD = 82816
_U1 = 15
_NFULL = 323
_BOXPAD = 3712
_OPAD = 384
_BPAD = 1280
_PPAD = 27648


def _prob_body(a_ref, b_ref, o_ref):
    s = jax.nn.sigmoid(a_ref[0])
    r = lax.dot_general(
        s, b_ref[0], (((1,), (1,)), ((), ())),
        preferred_element_type=jnp.float32)
    o_ref[0] = r


def _compute_prob(pred_logits, positive_maps):
    B, Q, D = pred_logits.shape
    C = positive_maps.shape[1]
    return pl.pallas_call(
        _prob_body,
        grid=(B,),
        in_specs=[
            pl.BlockSpec((1, Q, D), lambda b: (b, 0, 0)),
            pl.BlockSpec((1, C, D), lambda b: (b, 0, 0)),
        ],
        out_specs=pl.BlockSpec((1, Q, C), lambda b: (b, 0, 0)),
        out_shape=jax.ShapeDtypeStruct((B, Q, C), jnp.float32),
    )(pred_logits, positive_maps)


def _suffix_count_scan(hist, nbins, start_above, need):
    def cnt(b):
        return jnp.sum(hist[pl.ds(b * _LANES, _LANES)])

    def cond(st):
        b, above, c = st
        return jnp.logical_and(above + c < need, b > 0)

    def body(st):
        b, above, c = st
        return b - 1, above + c, cnt(b - 1)

    b0 = jnp.int32(nbins - 1)
    a0 = jnp.asarray(start_above, jnp.int32)
    b, above, _ = lax.while_loop(cond, body, (b0, a0, cnt(b0)))
    return b, above


def _sc_body(prob_hbm, boxes_hbm, scale_hbm,
             scores_hbm, labels_hbm, boxeso_hbm, probo_hbm,
             probv, hist, ck, ci, sk, si, bins2, cnts, bases,
             boxv, scalev, sbuf, lbuf, qbuf, bbuf, pchunk):
    nc = 2
    wid = lax.axis_index("s") * nc + lax.axis_index("c")
    lane = lax.iota(jnp.int32, _LANES)
    zeros_i = lane ^ lane
    ones = zeros_i + 1
    nvec = _N // _LANES
    c00, _ = plsc.scan_count(zeros_i)
    kbase = c00[0]

    for im in range(2):
        img = wid * 2 + im
        pltpu.sync_copy(prob_hbm.at[img], probv)
        pltpu.sync_copy(boxes_hbm.at[img], boxv)
        pltpu.sync_copy(scale_hbm.at[img], scalev)

        def zero_hist(j, _):
            for u in range(8):
                hist[pl.ds((j * 8 + u) * _LANES, _LANES)] = zeros_i
            return 0

        lax.fori_loop(0, _NBIN1 // 8, zero_hist, 0)

        def hist1(j, _):
            for u in range(_U1):
                b = lax.bitcast_convert_type(
                    probv[pl.ds((j * _U1 + u) * _LANES, _LANES)], jnp.int32)
                d1 = jnp.minimum(lax.shift_right_logical(b, 21), _NBIN1 - 1)
                plsc.addupdate_scatter(hist, [d1 * _LANES + lane], ones)
            return 0

        lax.fori_loop(0, nvec // _U1, hist1, 0)

        b1_bin, above1 = _suffix_count_scan(hist, _NBIN1, 0, _K)

        lax.fori_loop(0, _NBIN2 // 8, zero_hist, 0)

        def hist2(j, _):
            for u in range(_U1):
                b = lax.bitcast_convert_type(
                    probv[pl.ds((j * _U1 + u) * _LANES, _LANES)], jnp.int32)
                d1 = lax.shift_right_logical(b, 21)
                d2 = jnp.bitwise_and(lax.shift_right_logical(b, 13), _NBIN2 - 1)
                plsc.addupdate_scatter(hist, [d2 * _LANES + lane], ones,
                                       mask=d1 == b1_bin)
            return 0

        lax.fori_loop(0, nvec // _U1, hist2, 0)

        b2_bin, _above2 = _suffix_count_scan(hist, _NBIN2, above1, _K)
        thr = lax.shift_left(b1_bin * _NBIN2 + b2_bin, 13)

        last_lane = zeros_i + (_LANES - 1)

        def compact(j, off_splat):
            for u in range(_U1):
                jv = j * _U1 + u
                b = lax.bitcast_convert_type(
                    probv[pl.ds(jv * _LANES, _LANES)], jnp.int32)
                keep = b >= thr
                ki = jnp.where(keep, 1, 0)
                incl = plsc.cumsum(ki)
                pos = jnp.minimum(off_splat + incl - ki, _CAP - 1)
                plsc.store_scatter(ck, [pos], b, mask=keep)
                plsc.store_scatter(ci, [pos], jv * _LANES + lane, mask=keep)
                off_splat = off_splat + jnp.take(incl, last_lane)
            return off_splat

        moff = lax.fori_loop(0, nvec // _U1, compact, zeros_i)
        m = jnp.minimum(moff[0], _CAP - _LANES)
        ck[pl.ds(m, _LANES)] = zeros_i
        ci[pl.ds(m, _LANES)] = zeros_i
        mp = jnp.bitwise_and(m + _LANES - 1, ~(_LANES - 1))
        nv = mp // _LANES

        for p in range(4):
            src_k, src_i = (ck, ci) if p % 2 == 0 else (sk, si)
            dst_k, dst_i = (sk, si) if p % 2 == 0 else (ck, ci)
            shift = 8 * p

            def zero_bins(j, _):
                bins2[pl.ds(j * _LANES, _LANES)] = zeros_i
                return 0

            lax.fori_loop(0, _NBIN2 // _LANES, zero_bins, 0)

            def histp(j, _, src_k=src_k, shift=shift):
                k = src_k[pl.ds(j * _LANES, _LANES)]
                d = jnp.bitwise_and(lax.shift_right_logical(k, shift), 255)
                cnt, last = plsc.scan_count(d)
                plsc.addupdate_scatter(
                    bins2, [d], cnt - kbase + 1, mask=last)
                return 0

            lax.fori_loop(0, nv, histp, 0)

            def suffix(jj, carry):
                j = _NBIN2 // _LANES - 1 - jj
                sl = pl.ds(j * _LANES, _LANES)
                t = bins2[sl]
                incl = plsc.cumsum(t)
                total = jnp.take(incl, zeros_i + (_LANES - 1))
                bases[sl] = carry + total - incl
                return carry + total

            lax.fori_loop(0, _NBIN2 // _LANES, suffix, zeros_i)

            def permute(j, _, src_k=src_k, src_i=src_i,
                        dst_k=dst_k, dst_i=dst_i, shift=shift):
                k = src_k[pl.ds(j * _LANES, _LANES)]
                x = src_i[pl.ds(j * _LANES, _LANES)]
                d = jnp.bitwise_and(lax.shift_right_logical(k, shift), 255)
                base = plsc.load_gather(bases, [d])
                cnt, last = plsc.scan_count(d)
                pos = jnp.clip(base + cnt - kbase, 0, _CAP - 1)
                plsc.store_scatter(dst_k, [pos], k)
                plsc.store_scatter(dst_i, [pos], x)
                plsc.store_scatter(bases, [d], pos + 1, mask=last)
                return 0

            lax.fori_loop(0, nv, permute, 0)

        def finish(j, _):
            sl = pl.ds(j * _LANES, _LANES)
            k = ck[sl]
            x = ci[sl]
            sbuf[sl] = lax.bitcast_convert_type(k, jnp.float32)
            q = jnp.clip(x // _C, 0, _Q - 1)
            lbuf[sl] = x - q * _C
            qbuf[sl] = q
            return 0

        lax.fori_loop(0, _KPAD // _LANES, finish, 0)
        pltpu.sync_copy(sbuf, scores_hbm.at[img])
        pltpu.sync_copy(lbuf, labels_hbm.at[img])

        svec = scalev[pl.ds(0, _LANES)]
        rep4 = lax.shift_right_logical(lane, 2)
        sub4 = jnp.bitwise_and(lane, 3)
        low2 = jnp.bitwise_and(lane, ~2)
        hi2 = jnp.bitwise_or(lane, 2)
        sgn = jnp.where(jnp.bitwise_and(lane, 2) != 0,
                        jnp.float32(0.5), jnp.float32(-0.5))

        def boxes_body(j, _):
            qs = jnp.clip(plsc.load_gather(qbuf, [j * 4 + rep4]), 0, _Q - 1)
            vals = plsc.load_gather(boxv, [qs * 4 + sub4])
            c_at = jnp.take(vals, low2)
            s_at = jnp.take(vals, hi2)
            bbuf[pl.ds(j * _LANES, _LANES)] = (c_at + sgn * s_at) * svec
            return 0

        lax.fori_loop(0, _KPAD // 4, boxes_body, 0)
        pltpu.sync_copy(bbuf, boxeso_hbm.at[img])

        def prow_grp(g, base_slot):
            qv = jnp.clip(qbuf[pl.ds(base_slot + g * 4, _LANES)],
                          0, _Q - 1) * _C
            for u in range(4):
                qb = jnp.take(qv, zeros_i + u)
                for t in range(5):
                    pchunk[pl.ds((g * 4 + u) * _C + t * _LANES, _LANES)] = (
                        plsc.load_gather(probv, [qb + t * _LANES + lane]))
                pchunk[pl.ds((g * 4 + u) * _C + 76, _LANES)] = (
                    plsc.load_gather(probv, [qb + 76 + lane]))
            return base_slot

        for chunk in range(10):
            rows = 32 if chunk < 9 else 12
            lax.fori_loop(0, rows // 4, prow_grp, chunk * 32)
            nw = 2944 if chunk < 9 else 1152
            pltpu.sync_copy(
                pchunk.at[pl.ds(0, nw)],
                probo_hbm.at[img, pl.ds(chunk * 32 * _C, nw)])


@functools.partial(jax.jit, static_argnames=())
def _sc_stage(prob_flat, boxes_flat, scale16):
    B = prob_flat.shape[0]
    f32 = jnp.float32
    i32 = jnp.int32
    kern = pl.kernel(
        _sc_body,
        out_type=(
            jax.ShapeDtypeStruct((B, _OPAD), f32),
            jax.ShapeDtypeStruct((B, _OPAD), i32),
            jax.ShapeDtypeStruct((B, _BPAD), f32),
            jax.ShapeDtypeStruct((B, _PPAD), f32),
        ),
        mesh=plsc.VectorSubcoreMesh(core_axis_name="c", subcore_axis_name="s"),
        compiler_params=pltpu.CompilerParams(needs_layout_passes=False),
        scratch_types=[
            pltpu.VMEM((_NPAD,), f32),
            pltpu.VMEM((_NBIN1 * _LANES,), i32),
            pltpu.VMEM((_CAP,), i32),
            pltpu.VMEM((_CAP,), i32),
            pltpu.VMEM((_CAP,), i32),
            pltpu.VMEM((_CAP,), i32),
            pltpu.VMEM((_NBIN2,), i32),
            pltpu.VMEM((_NFULL * _LANES + _LANES,), i32),
            pltpu.VMEM((_NBIN2,), i32),
            pltpu.VMEM((_BOXPAD,), f32),
            pltpu.VMEM((128,), f32),
            pltpu.VMEM((_OPAD,), f32),
            pltpu.VMEM((_OPAD,), i32),
            pltpu.VMEM((_KPAD + _LANES,), i32),
            pltpu.VMEM((_BPAD,), f32),
            pltpu.VMEM((32 * _C,), f32),
        ],
    )
    return kern(prob_flat, boxes_flat, scale16)


def kernel(pred_logits, pred_boxes, target_sizes, positive_maps):
    B = pred_logits.shape[0]
    f32 = jnp.float32
    prob = _compute_prob(pred_logits, positive_maps)
    prob_flat = jnp.concatenate(
        [prob.reshape(B, _N), jnp.zeros((B, _NPAD - _N), f32)], axis=1)
    boxes_flat = jnp.concatenate(
        [pred_boxes.reshape(B, _Q * 4), jnp.zeros((B, _BOXPAD - _Q * 4), f32)],
        axis=1)
    img_h = target_sizes[:, 0]
    img_w = target_sizes[:, 1]
    scale16 = jnp.concatenate(
        [jnp.tile(jnp.stack([img_w, img_h, img_w, img_h], axis=1), (1, 4)),
         jnp.zeros((B, 112), f32)], axis=1)
    scores_p, labels_p, boxes_p, probo = _sc_stage(
        prob_flat, boxes_flat, scale16)
    scores = scores_p[:, :_K]
    labels = labels_p[:, :_K]
    boxes = boxes_p[:, :_K * 4].reshape(B, _K, 4)
    topk_prob = probo[:, :_K * _C].reshape(B, _K, _C)
    return scores, labels, boxes, topk_prob


# speedup vs baseline: 1.7828x; 1.7828x over previous
import functools

import jax
import jax.numpy as jnp
from jax import lax
from jax.experimental import pallas as pl
from jax.experimental.pallas import tpu as pltpu
from jax.experimental.pallas import tpu_sc as plsc

_K = 300
_Q = 900
_C = 92
_N = _Q * _C
_KPAD = 304
_CAP = 2048
_NBIN1 = 544
_NBIN2 = 256
_LANES = 16
_NPAD = 82816
_U1 = 15
_NFULL = 323
_BOXPAD = 3712
_OPAD = 384
_BPAD = 1280
_PPAD = 27648


def _prob_body(a_ref, b_ref, o_ref):
    s = jax.nn.sigmoid(a_ref[0])
    r = lax.dot_general(
        s, b_ref[0], (((1,), (1,)), ((), ())),
        preferred_element_type=jnp.float32)
    o_ref[0] = r


def _compute_prob(pred_logits, positive_maps):
    B, Q, D = pred_logits.shape
    C = positive_maps.shape[1]
    return pl.pallas_call(
        _prob_body,
        grid=(B,),
        in_specs=[
            pl.BlockSpec((1, Q, D), lambda b: (b, 0, 0)),
            pl.BlockSpec((1, C, D), lambda b: (b, 0, 0)),
        ],
        out_specs=pl.BlockSpec((1, Q, C), lambda b: (b, 0, 0)),
        out_shape=jax.ShapeDtypeStruct((B, Q, C), jnp.float32),
    )(pred_logits, positive_maps)


def _suffix_count_scan(hist, nbins, start_above, need):
    def cnt(b):
        return jnp.sum(hist[pl.ds(b * _LANES, _LANES)])

    def cond(st):
        b, above, c = st
        return jnp.logical_and(above + c < need, b > 0)

    def body(st):
        b, above, c = st
        return b - 1, above + c, cnt(b - 1)

    b0 = jnp.int32(nbins - 1)
    a0 = jnp.asarray(start_above, jnp.int32)
    b, above, _ = lax.while_loop(cond, body, (b0, a0, cnt(b0)))
    return b, above


def _sc_body(prob_hbm, boxes_hbm, scale_hbm,
             scores_hbm, labels_hbm, boxeso_hbm, probo_hbm,
             probv, hist, ck, ci, sk, si, bins2, cnts, bases,
             boxv, scalev, sbuf, lbuf, qbuf, bbuf, pchunk):
    nc = 2
    wid = lax.axis_index("s") * nc + lax.axis_index("c")
    lane = lax.iota(jnp.int32, _LANES)
    zeros_i = lane ^ lane
    ones = zeros_i + 1
    nvec = _N // _LANES
    c00, _ = plsc.scan_count(zeros_i)
    kbase = c00[0]

    for im in range(2):
        img = wid * 2 + im
        pltpu.sync_copy(prob_hbm.at[img], probv)
        pltpu.sync_copy(boxes_hbm.at[img], boxv)
        pltpu.sync_copy(scale_hbm.at[img], scalev)

        def zero_hist(j, _):
            for u in range(8):
                hist[pl.ds((j * 8 + u) * _LANES, _LANES)] = zeros_i
            return 0

        lax.fori_loop(0, _NBIN1 // 8, zero_hist, 0)

        def hist1(j, _):
            bs = [lax.bitcast_convert_type(
                probv[pl.ds((j * _U1 + u) * _LANES, _LANES)], jnp.int32)
                for u in range(_U1)]
            slots = [jnp.minimum(lax.shift_right_logical(b, 21), _NBIN1 - 1)
                     * _LANES + lane for b in bs]
            for u in range(_U1):
                plsc.addupdate_scatter(hist, [slots[u]], ones)
            return 0

        lax.fori_loop(0, nvec // _U1, hist1, 0)

        b1_bin, above1 = _suffix_count_scan(hist, _NBIN1, 0, _K)

        lax.fori_loop(0, _NBIN2 // 8, zero_hist, 0)

        def hist2(j, _):
            bs = [lax.bitcast_convert_type(
                probv[pl.ds((j * _U1 + u) * _LANES, _LANES)], jnp.int32)
                for u in range(_U1)]
            slots = [jnp.bitwise_and(lax.shift_right_logical(b, 13),
                                     _NBIN2 - 1) * _LANES + lane for b in bs]
            masks = [lax.shift_right_logical(b, 21) == b1_bin for b in bs]
            for u in range(_U1):
                plsc.addupdate_scatter(hist, [slots[u]], ones, mask=masks[u])
            return 0

        lax.fori_loop(0, nvec // _U1, hist2, 0)

        b2_bin, _above2 = _suffix_count_scan(hist, _NBIN2, above1, _K)
        thr = lax.shift_left(b1_bin * _NBIN2 + b2_bin, 13)

        last_lane = zeros_i + (_LANES - 1)

        def compact(j, off_splat):
            _UC = 5
            bs = [lax.bitcast_convert_type(
                probv[pl.ds((j * _UC + u) * _LANES, _LANES)], jnp.int32)
                for u in range(_UC)]
            keeps = [b >= thr for b in bs]
            kis = [jnp.where(k, 1, 0) for k in keeps]
            incls = [plsc.cumsum(ki) for ki in kis]
            excls = [i - ki for i, ki in zip(incls, kis)]
            totals = [jnp.take(i, last_lane) for i in incls]
            for u in range(_UC):
                pos = jnp.minimum(off_splat + excls[u], _CAP - 1)
                plsc.store_scatter(ck, [pos], bs[u], mask=keeps[u])
                plsc.store_scatter(
                    ci, [pos], (j * _UC + u) * _LANES + lane, mask=keeps[u])
                off_splat = off_splat + totals[u]
            return off_splat

        moff = lax.fori_loop(0, nvec // 5, compact, zeros_i)
        m = jnp.minimum(moff[0], _CAP - _LANES)
        ck[pl.ds(m, _LANES)] = zeros_i
        ci[pl.ds(m, _LANES)] = zeros_i
        mp = jnp.bitwise_and(m + _LANES - 1, ~(_LANES - 1))
        nv = mp // _LANES

        for p in range(4):
            src_k, src_i = (ck, ci) if p % 2 == 0 else (sk, si)
            dst_k, dst_i = (sk, si) if p % 2 == 0 else (ck, ci)
            shift = 8 * p

            def zero_bins(j, _):
                bins2[pl.ds(j * _LANES, _LANES)] = zeros_i
                return 0

            lax.fori_loop(0, _NBIN2 // _LANES, zero_bins, 0)

            def histp(j, _, src_k=src_k, shift=shift):
                k = src_k[pl.ds(j * _LANES, _LANES)]
                d = jnp.bitwise_and(lax.shift_right_logical(k, shift), 255)
                cnt, last = plsc.scan_count(d)
                plsc.addupdate_scatter(
                    bins2, [d], cnt - kbase + 1, mask=last)
                return 0

            lax.fori_loop(0, nv, histp, 0)

            def suffix(jj, carry):
                j = _NBIN2 // _LANES - 1 - jj
                sl = pl.ds(j * _LANES, _LANES)
                t = bins2[sl]
                incl = plsc.cumsum(t)
                total = jnp.take(incl, zeros_i + (_LANES - 1))
                bases[sl] = carry + total - incl
                return carry + total

            lax.fori_loop(0, _NBIN2 // _LANES, suffix, zeros_i)

            def permute(j, _, src_k=src_k, src_i=src_i,
                        dst_k=dst_k, dst_i=dst_i, shift=shift):
                k = src_k[pl.ds(j * _LANES, _LANES)]
                x = src_i[pl.ds(j * _LANES, _LANES)]
                d = jnp.bitwise_and(lax.shift_right_logical(k, shift), 255)
                base = plsc.load_gather(bases, [d])
                cnt, last = plsc.scan_count(d)
                pos = jnp.clip(base + cnt - kbase, 0, _CAP - 1)
                plsc.store_scatter(dst_k, [pos], k)
                plsc.store_scatter(dst_i, [pos], x)
                plsc.store_scatter(bases, [d], pos + 1, mask=last)
                return 0

            lax.fori_loop(0, nv, permute, 0)

        ks = [ck[pl.ds(j * _LANES, _LANES)]
              for j in range(_KPAD // _LANES)]
        xs = [ci[pl.ds(j * _LANES, _LANES)]
              for j in range(_KPAD // _LANES)]
        qs_all = [jnp.clip(x // _C, 0, _Q - 1) for x in xs]
        for j in range(_KPAD // _LANES):
            sl = pl.ds(j * _LANES, _LANES)
            sbuf[sl] = lax.bitcast_convert_type(ks[j], jnp.float32)
            lbuf[sl] = xs[j] - qs_all[j] * _C
            qbuf[sl] = qs_all[j]
        pltpu.sync_copy(sbuf, scores_hbm.at[img])
        pltpu.sync_copy(lbuf, labels_hbm.at[img])

        svec = scalev[pl.ds(0, _LANES)]
        rep4 = lax.shift_right_logical(lane, 2)
        sub4 = jnp.bitwise_and(lane, 3)
        low2 = jnp.bitwise_and(lane, ~2)
        hi2 = jnp.bitwise_or(lane, 2)
        sgn = jnp.where(jnp.bitwise_and(lane, 2) != 0,
                        jnp.float32(0.5), jnp.float32(-0.5))

        def boxes_body(j, _):
            qsl = [jnp.clip(plsc.load_gather(qbuf, [(j * 4 + u) * 4 + rep4]),
                            0, _Q - 1) for u in range(4)]
            valsl = [plsc.load_gather(boxv, [q * 4 + sub4]) for q in qsl]
            outs = [(jnp.take(v, low2) + sgn * jnp.take(v, hi2)) * svec
                    for v in valsl]
            for u in range(4):
                bbuf[pl.ds((j * 4 + u) * _LANES, _LANES)] = outs[u]
            return 0

        lax.fori_loop(0, _KPAD // (4 * 4), boxes_body, 0)
        pltpu.sync_copy(bbuf, boxeso_hbm.at[img])

        def prow_grp(g, base_slot):
            qv = jnp.clip(qbuf[pl.ds(base_slot + g * 4, _LANES)],
                          0, _Q - 1) * _C
            qbs = [jnp.take(qv, zeros_i + u) for u in range(4)]
            offs = [t * _LANES for t in range(5)] + [76]
            vals = [plsc.load_gather(probv, [qb + o + lane])
                    for qb in qbs for o in offs]
            for u in range(4):
                for t in range(6):
                    pchunk[pl.ds((g * 4 + u) * _C + offs[t], _LANES)] = (
                        vals[u * 6 + t])
            return base_slot

        for chunk in range(10):
            rows = 32 if chunk < 9 else 12
            lax.fori_loop(0, rows // 4, prow_grp, chunk * 32)
            nw = 2944 if chunk < 9 else 1152
            pltpu.sync_copy(
                pchunk.at[pl.ds(0, nw)],
                probo_hbm.at[img, pl.ds(chunk * 32 * _C, nw)])


@functools.partial(jax.jit, static_argnames=())
def _sc_stage(prob_flat, boxes_flat, scale16):
    B = prob_flat.shape[0]
    f32 = jnp.float32
    i32 = jnp.int32
    kern = pl.kernel(
        _sc_body,
        out_type=(
            jax.ShapeDtypeStruct((B, _OPAD), f32),
            jax.ShapeDtypeStruct((B, _OPAD), i32),
            jax.ShapeDtypeStruct((B, _BPAD), f32),
            jax.ShapeDtypeStruct((B, _PPAD), f32),
        ),
        mesh=plsc.VectorSubcoreMesh(core_axis_name="c", subcore_axis_name="s"),
        compiler_params=pltpu.CompilerParams(needs_layout_passes=False),
        scratch_types=[
            pltpu.VMEM((_NPAD,), f32),
            pltpu.VMEM((_NBIN1 * _LANES,), i32),
            pltpu.VMEM((_CAP,), i32),
            pltpu.VMEM((_CAP,), i32),
            pltpu.VMEM((_CAP,), i32),
            pltpu.VMEM((_CAP,), i32),
            pltpu.VMEM((_NBIN2,), i32),
            pltpu.VMEM((_NFULL * _LANES + _LANES,), i32),
            pltpu.VMEM((_NBIN2,), i32),
            pltpu.VMEM((_BOXPAD,), f32),
            pltpu.VMEM((128,), f32),
            pltpu.VMEM((_OPAD,), f32),
            pltpu.VMEM((_OPAD,), i32),
            pltpu.VMEM((_KPAD + _LANES,), i32),
            pltpu.VMEM((_BPAD,), f32),
            pltpu.VMEM((32 * _C,), f32),
        ],
    )
    return kern(prob_flat, boxes_flat, scale16)


def kernel(pred_logits, pred_boxes, target_sizes, positive_maps):
    B = pred_logits.shape[0]
    f32 = jnp.float32
    prob = _compute_prob(pred_logits, positive_maps)
    prob_flat = jnp.concatenate(
        [prob.reshape(B, _N), jnp.zeros((B, _NPAD - _N), f32)], axis=1)
    boxes_flat = jnp.concatenate(
        [pred_boxes.reshape(B, _Q * 4), jnp.zeros((B, _BOXPAD - _Q * 4), f32)],
        axis=1)
    img_h = target_sizes[:, 0]
    img_w = target_sizes[:, 1]
    scale16 = jnp.concatenate(
        [jnp.tile(jnp.stack([img_w, img_h, img_w, img_h], axis=1), (1, 4)),
         jnp.zeros((B, 112), f32)], axis=1)
    scores_p, labels_p, boxes_p, probo = _sc_stage(
        prob_flat, boxes_flat, scale16)
    scores = scores_p[:, :_K]
    labels = labels_p[:, :_K]
    boxes = boxes_p[:, :_K * 4].reshape(B, _K, 4)
    topk_prob = probo[:, :_K * _C].reshape(B, _K, _C)
    return scores, labels, boxes, topk_prob
